# baseline (speedup 1.0000x reference)
import functools

import jax
import jax.numpy as jnp
from jax import lax
from jax.experimental import pallas as pl
from jax.experimental.pallas import tpu as pltpu
from jax.experimental.pallas import tpu_sc as plsc

B = 4096
HIST = 200
D = 256
NCLASS = 1000

NC = 2
NS = 16
NW = NC * NS
ROWS_PER_W = B // NW
HIST_PAD = 208
CHUNK = HIST_PAD // 2
DC = D // 16


def _pool_body(x_hbm, table_hbm, h0_hbm, idx_v, buf0, buf1, stage, sem0, sem1):
    wid = lax.axis_index("s") * NC + lax.axis_index("c")

    pltpu.sync_copy(x_hbm.at[wid], idx_v)

    def fire(c, buf, sem):
        pltpu.async_copy(table_hbm.at[idx_v.at[c]], buf, sem)

    def drain(buf, sem):
        pltpu.make_async_copy(table_hbm.at[pl.ds(0, CHUNK)], buf, sem).wait()

    def reduce_chunk(buf, carry):
        def body(l, carry):
            sums, cnts = carry
            new_s, new_c = [], []
            for dc in range(DC):
                v = buf[l, pl.ds(dc * 16, 16)]
                new_s.append(sums[dc] + v)
                new_c.append(cnts[dc] + jnp.where(v != 0.0, 1.0, 0.0))
            return tuple(new_s), tuple(new_c)
        return lax.fori_loop(0, CHUNK, body, carry)

    fire(0, buf0, sem0)
    fire(1, buf1, sem1)

    zeros = tuple(jnp.zeros((16,), jnp.float32) for _ in range(DC))

    def row_body(r, _):
        drain(buf0, sem0)
        carry = reduce_chunk(buf0, (zeros, zeros))

        @pl.when(r < ROWS_PER_W - 1)
        def _():
            fire(2 * (r + 1), buf0, sem0)

        drain(buf1, sem1)
        sums, cnts = reduce_chunk(buf1, carry)

        @pl.when(r < ROWS_PER_W - 1)
        def _():
            fire(2 * (r + 1) + 1, buf1, sem1)

        rm = lax.rem(r, 8)
        for dc in range(DC):
            h = sums[dc] / jnp.maximum(cnts[dc], 1.0)
            stage[rm, pl.ds(dc * 16, 16)] = h

        @pl.when(rm == 7)
        def _():
            base = pl.multiple_of(wid * ROWS_PER_W + r - 7, 8)
            pltpu.sync_copy(stage, h0_hbm.at[pl.ds(base, 8)])

        return 0

    lax.fori_loop(0, ROWS_PER_W, row_body, 0)


def _pool(x_r, table):
    return pl.kernel(
        _pool_body,
        mesh=plsc.VectorSubcoreMesh(core_axis_name="c", subcore_axis_name="s"),
        out_type=jax.ShapeDtypeStruct((B, D), jnp.float32),
        scratch_types=[
            pltpu.VMEM((2 * ROWS_PER_W, CHUNK), jnp.int32),
            pltpu.VMEM((CHUNK, D), jnp.float32),
            pltpu.VMEM((CHUNK, D), jnp.float32),
            pltpu.VMEM((8, D), jnp.float32),
            pltpu.SemaphoreType.DMA,
            pltpu.SemaphoreType.DMA,
        ],
    )(x_r, table)


def _mlp_body(h0_ref, W1_ref, b1_ref, W2_ref, b2_ref, out_ref):
    h1 = jnp.dot(h0_ref[...], W1_ref[...], preferred_element_type=jnp.float32)
    h1 = jnp.maximum(h1 + b1_ref[...], 0.0)
    o = jnp.dot(h1, W2_ref[...], preferred_element_type=jnp.float32)
    out_ref[...] = jnp.maximum(o + b2_ref[...], 0.0)


def _mlp(h0, W1, b1, W2, b2):
    bm = 512
    return pl.pallas_call(
        _mlp_body,
        grid=(B // bm,),
        in_specs=[
            pl.BlockSpec((bm, D), lambda i: (i, 0)),
            pl.BlockSpec((D, 128), lambda i: (0, 0)),
            pl.BlockSpec((1, 128), lambda i: (0, 0)),
            pl.BlockSpec((128, NCLASS), lambda i: (0, 0)),
            pl.BlockSpec((1, NCLASS), lambda i: (0, 0)),
        ],
        out_specs=pl.BlockSpec((bm, NCLASS), lambda i: (i, 0)),
        out_shape=jax.ShapeDtypeStruct((B, NCLASS), jnp.float32),
    )(h0, W1, b1.reshape(1, -1), W2, b2.reshape(1, -1))


def kernel(x, table, W1, b1, W2, b2):
    x_pad = jnp.pad(x, ((0, 0), (0, HIST_PAD - HIST)))
    x_r = x_pad.reshape(NW, 2 * ROWS_PER_W, CHUNK)
    h0 = _pool(x_r, table)
    return _mlp(h0, W1, b1, W2, b2)
